# baseline (speedup 1.0000x reference)
import functools

import jax
import jax.numpy as jnp
from jax import lax
from jax.experimental import pallas as pl
from jax.experimental.pallas import tpu as pltpu
from jax.experimental.pallas import tpu_sc as plsc

_D = 128
_T = 4
_B, _N = 8, 64
_NEG = 0.2
_LANES = 16
_NC, _NS = 2, 16
_ROWS = (_B * _N) // (_NC * _NS)
_TPB = _N // _ROWS
_PAD = 128


def _sc_body(src_hbm, dst_hbm, edges_hbm, a_hbm, out_hbm,
             src_v, dst_v, a_v, e_v, sp_v, dp_v, o_v):
    c = lax.axis_index("c")
    s = lax.axis_index("s")
    b = c * (_NS // _TPB) + s // _TPB
    i0 = (s % _TPB) * _ROWS

    pltpu.sync_copy(src_hbm.at[b, pl.ds(i0, _ROWS), :], src_v)
    pltpu.sync_copy(dst_hbm.at[b], dst_v)
    pltpu.sync_copy(a_hbm, a_v)
    pltpu.sync_copy(edges_hbm.at[b, pl.ds(i0, _ROWS), :], e_v)

    lane = lax.iota(jnp.int32, _LANES)
    groups = _N // _LANES

    acc_sp = [jnp.zeros((_LANES,), jnp.float32) for _ in range(_T)]
    acc_dp = [[jnp.zeros((_LANES,), jnp.float32) for _ in range(groups)]
              for _ in range(_T)]
    for t in range(_D):
        t_splat = jnp.full((_LANES,), t, jnp.int32)
        col_s = plsc.load_gather(src_v, [lane, t_splat])
        col_d = [plsc.load_gather(dst_v, [lane + g * _LANES, t_splat])
                 for g in range(groups)]
        for k in range(_T):
            a_s = plsc.load_gather(
                a_v, [jnp.full((_LANES,), _PAD + k * 2 * _D + t, jnp.int32)])
            a_d = plsc.load_gather(
                a_v, [jnp.full((_LANES,), _PAD + k * 2 * _D + _D + t,
                               jnp.int32)])
            acc_sp[k] = acc_sp[k] + col_s * a_s
            for g in range(groups):
                acc_dp[k][g] = acc_dp[k][g] + col_d[g] * a_d

    for k in range(_T):
        plsc.store_scatter(sp_v, [lane * _T + k], acc_sp[k])
        for g in range(groups):
            plsc.store_scatter(
                dp_v, [lane * _T + (g * _LANES * _T + k)], acc_dp[k][g])

    for i in range(_ROWS):
        for cch in range(groups):
            e = e_v[i, pl.ds(cch * _LANES, _LANES)]
            dpg = plsc.load_gather(dp_v, [lane * _T + (cch * _LANES * _T) + e])
            spg = plsc.load_gather(sp_v, [e + i * _T])
            sv = spg + dpg
            o_v[i, pl.ds(cch * _LANES, _LANES)] = jnp.where(
                sv >= 0.0, sv, _NEG * sv)

    pltpu.sync_copy(o_v, out_hbm.at[b, pl.ds(i0, _ROWS), :])


@functools.cache
def _sc_kernel():
    return functools.partial(
        pl.kernel,
        out_type=jax.ShapeDtypeStruct((_B, _N, _N), jnp.float32),
        mesh=plsc.VectorSubcoreMesh(core_axis_name="c", subcore_axis_name="s",
                                    num_cores=_NC, num_subcores=_NS),
        compiler_params=pltpu.CompilerParams(needs_layout_passes=False),
        scratch_types=[
            pltpu.VMEM((_ROWS, _D), jnp.float32),
            pltpu.VMEM((_N, _D), jnp.float32),
            pltpu.VMEM((_PAD + 2 * _D * _T,), jnp.float32),
            pltpu.VMEM((_ROWS, _N), jnp.int32),
            pltpu.VMEM((_ROWS * _T,), jnp.float32),
            pltpu.VMEM((_N * _T,), jnp.float32),
            pltpu.VMEM((_ROWS, _N), jnp.float32),
        ],
    )(_sc_body)


def kernel(src_embeddings, dst_embeddings, edges, a):
    a_pad = jnp.concatenate(
        [jnp.zeros((_PAD,), jnp.float32), a.reshape(2 * _D * _T)])
    return _sc_kernel()(src_embeddings, dst_embeddings,
                        edges.astype(jnp.int32), a_pad)


# speedup vs baseline: 1.7425x; 1.7425x over previous
import functools

import jax
import jax.numpy as jnp
from jax import lax
from jax.experimental import pallas as pl
from jax.experimental.pallas import tpu as pltpu
from jax.experimental.pallas import tpu_sc as plsc

_D = 128
_T = 4
_B, _N = 8, 64
_NEG = 0.2
_LANES = 16
_NC, _NS = 2, 16
_ROWS = (_B * _N) // (_NC * _NS)
_TPB = _N // _ROWS


def _tc_proj_body(src_ref, dst_ref, a_ref, ys_ref, yd_ref):
    dn = (((1,), (1,)), ((), ()))
    a8 = a_ref[...]
    ys_ref[...] = lax.dot_general(src_ref[...], a8, dn,
                                  preferred_element_type=jnp.float32)
    yd_ref[...] = lax.dot_general(dst_ref[...], a8, dn,
                                  preferred_element_type=jnp.float32)


def _tc_proj(src2, dst2, a8):
    r = src2.shape[0]
    return pl.pallas_call(
        _tc_proj_body,
        out_shape=(
            jax.ShapeDtypeStruct((r, 2 * _T), jnp.float32),
            jax.ShapeDtypeStruct((r, 2 * _T), jnp.float32),
        ),
    )(src2, dst2, a8)


def _sc_gather_body(ys_hbm, yd_hbm, edges_hbm, out_hbm, sp_v, dp_v, e_v, o_v):
    c = lax.axis_index("c")
    s = lax.axis_index("s")
    b = c * (_NS // _TPB) + s // _TPB
    i0 = (s % _TPB) * _ROWS

    pltpu.sync_copy(ys_hbm.at[pl.ds(b * _N + i0, _ROWS), :], sp_v)
    pltpu.sync_copy(yd_hbm.at[pl.ds(b * _N, _N), :], dp_v)
    pltpu.sync_copy(edges_hbm.at[b, pl.ds(i0, _ROWS), :], e_v)

    lane = lax.iota(jnp.int32, _LANES)
    for i in range(_ROWS):
        i_splat = jnp.full((_LANES,), i, jnp.int32)
        for cch in range(_N // _LANES):
            e = e_v[i, pl.ds(cch * _LANES, _LANES)]
            e2 = e + e
            dpg = plsc.load_gather(dp_v, [lane + cch * _LANES, e2 + 1])
            spg = plsc.load_gather(sp_v, [i_splat, e2])
            sv = spg + dpg
            o_v[i, pl.ds(cch * _LANES, _LANES)] = jnp.where(
                sv >= 0.0, sv, _NEG * sv)

    pltpu.sync_copy(o_v, out_hbm.at[b, pl.ds(i0, _ROWS), :])


@functools.cache
def _sc_gather():
    return functools.partial(
        pl.kernel,
        out_type=jax.ShapeDtypeStruct((_B, _N, _N), jnp.float32),
        mesh=plsc.VectorSubcoreMesh(core_axis_name="c", subcore_axis_name="s",
                                    num_cores=_NC, num_subcores=_NS),
        compiler_params=pltpu.CompilerParams(needs_layout_passes=False),
        scratch_types=[
            pltpu.VMEM((_ROWS, 2 * _T), jnp.float32),
            pltpu.VMEM((_N, 2 * _T), jnp.float32),
            pltpu.VMEM((_ROWS, _N), jnp.int32),
            pltpu.VMEM((_ROWS, _N), jnp.float32),
        ],
    )(_sc_gather_body)


def kernel(src_embeddings, dst_embeddings, edges, a):
    b, n, d = src_embeddings.shape
    ys, yd = _tc_proj(src_embeddings.reshape(b * n, d),
                      dst_embeddings.reshape(b * n, d),
                      a.reshape(2 * _T, d))
    return _sc_gather()(ys, yd, edges.astype(jnp.int32))


# speedup vs baseline: 1.8284x; 1.0493x over previous
import functools

import jax
import jax.numpy as jnp
from jax import lax
from jax.experimental import pallas as pl
from jax.experimental.pallas import tpu as pltpu
from jax.experimental.pallas import tpu_sc as plsc

_D = 128
_T = 4
_B, _N = 8, 64
_NEG = 0.2
_LANES = 16
_NC, _NS = 2, 16
_ROWS = (_B * _N) // (_NC * _NS)
_TPB = _N // _ROWS


def _tc_proj_body(src_ref, dst_ref, a_ref, ys_ref, yd_ref):
    dn = (((1,), (1,)), ((), ()))
    a8 = a_ref[...]
    ys_ref[...] = lax.dot_general(src_ref[...], a8, dn,
                                  preferred_element_type=jnp.float32)
    yd_ref[...] = lax.dot_general(dst_ref[...], a8, dn,
                                  preferred_element_type=jnp.float32)


def _tc_proj(src2, dst2, a8):
    r = src2.shape[0]
    return pl.pallas_call(
        _tc_proj_body,
        out_shape=(
            jax.ShapeDtypeStruct((r, 2 * _T), jnp.float32),
            jax.ShapeDtypeStruct((r, 2 * _T), jnp.float32),
        ),
    )(src2, dst2, a8)


def _sc_gather_body(ys_hbm, yd_hbm, edges_hbm, out_hbm,
                    sp_v, dp_v, e_v, o_v, sem1, sem2, sem3):
    c = lax.axis_index("c")
    s = lax.axis_index("s")
    b = c * (_NS // _TPB) + s // _TPB
    i0 = (s % _TPB) * _ROWS

    cp1 = pltpu.async_copy(ys_hbm.at[pl.ds(b * _N + i0, _ROWS), :], sp_v, sem1)
    cp2 = pltpu.async_copy(yd_hbm.at[pl.ds(b * _N, _N), :], dp_v, sem2)
    cp3 = pltpu.async_copy(edges_hbm.at[b, pl.ds(i0, _ROWS), :], e_v, sem3)
    cp1.wait()
    cp2.wait()
    cp3.wait()

    lane = lax.iota(jnp.int32, _LANES)
    for i in range(_ROWS):
        i_splat = jnp.full((_LANES,), i, jnp.int32)
        for cch in range(_N // _LANES):
            e = e_v[i, pl.ds(cch * _LANES, _LANES)]
            e2 = e + e
            dpg = plsc.load_gather(dp_v, [lane + cch * _LANES, e2 + 1])
            spg = plsc.load_gather(sp_v, [i_splat, e2])
            sv = spg + dpg
            o_v[i, pl.ds(cch * _LANES, _LANES)] = jnp.where(
                sv >= 0.0, sv, _NEG * sv)

    pltpu.sync_copy(o_v, out_hbm.at[b, pl.ds(i0, _ROWS), :])


@functools.cache
def _sc_gather():
    return functools.partial(
        pl.kernel,
        out_type=jax.ShapeDtypeStruct((_B, _N, _N), jnp.float32),
        mesh=plsc.VectorSubcoreMesh(core_axis_name="c", subcore_axis_name="s",
                                    num_cores=_NC, num_subcores=_NS),
        compiler_params=pltpu.CompilerParams(needs_layout_passes=False),
        scratch_types=[
            pltpu.VMEM((_ROWS, 2 * _T), jnp.float32),
            pltpu.VMEM((_N, 2 * _T), jnp.float32),
            pltpu.VMEM((_ROWS, _N), jnp.int32),
            pltpu.VMEM((_ROWS, _N), jnp.float32),
            pltpu.SemaphoreType.DMA,
            pltpu.SemaphoreType.DMA,
            pltpu.SemaphoreType.DMA,
        ],
    )(_sc_gather_body)


def kernel(src_embeddings, dst_embeddings, edges, a):
    b, n, d = src_embeddings.shape
    ys, yd = _tc_proj(src_embeddings.reshape(b * n, d),
                      dst_embeddings.reshape(b * n, d),
                      a.reshape(2 * _T, d))
    return _sc_gather()(ys, yd, edges.astype(jnp.int32))
